# baseline (speedup 1.0000x reference)
import jax
import jax.numpy as jnp
from jax import lax
from jax.experimental import pallas as pl
from jax.experimental.pallas import tpu as pltpu
from jax.experimental.pallas import tpu_sc as plsc

_VOCAB = 1000000
_DIM = 64
_SEQ = 200
_BATCH = 4096

_NC, _NS = 2, 16
_NW = _NC * _NS
_ROWS = _BATCH * _SEQ
_RPW = _ROWS // _NW
_CH = 256
_NSUB = _CH // 128
_NB = 2
_NCHUNK = _RPW // _CH
_PER = 25


def _body(x_hbm, table_hbm, pos_hbm, sidx_hbm, out_hbm,
          idx_v, sidx_v, buf_v, pos_sh, g0, g1, w0, w1, psem):
    gsems, wsems = [g0, g1], [w0, w1]
    sid = lax.axis_index("s")
    wid = sid * _NC + lax.axis_index("c")

    @pl.when(sid == 0)
    def _fill_pos():
        pltpu.sync_copy(pos_hbm, pos_sh)

    pltpu.sync_copy(sidx_hbm, sidx_v)
    plsc.subcore_barrier()

    def wait_write(b):
        pltpu.make_async_copy(buf_v.at[b], out_hbm.at[pl.ds(0, _CH)], wsems[b]).wait()

    def step(k, b, recycle):
        base = wid * _RPW + k * _CH
        m = lax.rem(k, _PER)
        pltpu.sync_copy(x_hbm.at[pl.ds(base, _CH)], idx_v.at[b])
        if recycle:
            wait_write(b)
        gd = [
            pltpu.async_copy(
                table_hbm.at[idx_v.at[b, pl.ds(j * 128, 128)]],
                buf_v.at[b, pl.ds(j * 128, 128)],
                gsems[b],
            )
            for j in range(_NSUB)
        ]
        for d in gd:
            d.wait()
        pd = [
            pltpu.async_copy(
                pos_sh.at[sidx_v.at[pl.ds(m * _CH + j * 128, 128)]],
                buf_v.at[b, pl.ds(j * 128, 128)],
                psem,
                add=True,
            )
            for j in range(_NSUB)
        ]
        for d in pd:
            d.wait()
        pltpu.async_copy(buf_v.at[b], out_hbm.at[pl.ds(base, _CH)], wsems[b])

    for db in range(_NB):
        step(db, db, recycle=False)

    @pl.loop(1, _NCHUNK // _NB)
    def _grp(g):
        for db in range(_NB):
            step(g * _NB + db, db, recycle=True)

    for db in range(_NB):
        wait_write(db)


def kernel(x, table, pos):
    xf = x.reshape(_ROWS)
    sidx = jnp.arange(_PER * _CH, dtype=jnp.int32) % _SEQ
    run = pl.kernel(
        _body,
        out_type=jax.ShapeDtypeStruct((_ROWS, _DIM), jnp.float32),
        mesh=plsc.VectorSubcoreMesh(core_axis_name="c", subcore_axis_name="s"),
        scratch_types=[
            pltpu.VMEM((_NB, _CH), jnp.int32),
            pltpu.VMEM((_PER * _CH,), jnp.int32),
            pltpu.VMEM((_NB, _CH, _DIM), jnp.float32),
            pltpu.VMEM_SHARED((_SEQ, _DIM), jnp.float32),
        ] + [pltpu.SemaphoreType.DMA] * (2 * _NB + 1),
        compiler_params=pltpu.CompilerParams(use_tc_tiling_on_sc=False),
    )
    out = run(xf, table, pos, sidx)
    return out.reshape(_BATCH, _SEQ, _DIM)
